# baseline (speedup 1.0000x reference)
import functools

import jax
import jax.numpy as jnp
from jax import lax
from jax.experimental import pallas as pl
from jax.experimental.pallas import tpu as pltpu
from jax.experimental.pallas import tpu_sc as plsc

_T, _D, _E = 8192, 2048, 64
_NC, _NS, _L = 2, 16, 16
_NW = _NC * _NS
_ROWS_W = _T // _NW
_GROUPS_W = _ROWS_W // _L
_LT = 128
_TILES_W = _ROWS_W // _LT


def _logits_block(x_ref, w_ref, logits_t_ref, load_ref):
    x = x_ref[...]
    w = w_ref[...]
    logits = lax.dot_general(
        x, w, (((1,), (1,)), ((), ())),
        preferred_element_type=jnp.float32)
    bt = logits.shape[0]
    logits_t_ref[...] = logits.T.reshape(_E, bt // _LT, _LT)
    m = jnp.max(logits, axis=1, keepdims=True)
    ex = jnp.exp(logits - m)
    load_ref[...] = ex / jnp.sum(ex, axis=1, keepdims=True)


def _tc_stage(x, w_gate, block_t=2048):
    t, d = x.shape
    e = w_gate.shape[0]
    bt_tiles = block_t // _LT
    return pl.pallas_call(
        _logits_block,
        grid=(t // block_t,),
        in_specs=[
            pl.BlockSpec((block_t, d), lambda i: (i, 0)),
            pl.BlockSpec((e, d), lambda i: (0, 0)),
        ],
        out_specs=[
            pl.BlockSpec((e, bt_tiles, _LT), lambda i: (0, i, 0)),
            pl.BlockSpec((block_t, e), lambda i: (i, 0)),
        ],
        out_shape=[
            jax.ShapeDtypeStruct((e, t // _LT, _LT), jnp.float32),
            jax.ShapeDtypeStruct((t, e), jnp.float32),
        ],
    )(x, w_gate)


def _sc_gates_body(logits_t_hbm, gates_hbm, lt, gv):
    c = lax.axis_index("c")
    s = lax.axis_index("s")
    wid = s * _NC + c
    tbase = wid * _ROWS_W

    pltpu.sync_copy(
        logits_t_hbm.at[:, pl.ds(wid * _TILES_W, _TILES_W), :], lt)

    zeros = jnp.zeros((_L,), jnp.float32)
    iota = lax.iota(jnp.int32, _L)
    minf = jnp.full((_L,), -jnp.inf, jnp.float32)
    izero = jnp.zeros((_L,), jnp.int32)

    def _group(g, carry):
        t0 = pl.multiple_of(g * _L, _L)
        tile = g // (_LT // _L)
        toff = pl.multiple_of((g % (_LT // _L)) * _L, _L)
        trow = t0 + iota

        for k in range(_L * _E // _L):
            gv[t0 + k // (_E // _L), pl.ds((k % (_E // _L)) * _L, _L)] = zeros

        n_chains = 4
        span = _E // n_chains
        tops = []
        for ch in range(n_chains):
            m1, i1, m2, i2 = minf, izero, minf, izero
            for j in range(span):
                e_i = ch * span + j
                v = lt[e_i, tile, pl.ds(toff, _L)]
                ev = jnp.full((_L,), e_i, jnp.int32)
                gt1 = v > m1
                gt2 = v > m2
                m2n = jnp.where(gt1, m1, jnp.where(gt2, v, m2))
                i2n = jnp.where(gt1, i1, jnp.where(gt2, ev, i2))
                m1 = jnp.where(gt1, v, m1)
                i1 = jnp.where(gt1, ev, i1)
                m2, i2 = m2n, i2n
            tops.append((m1, i1, m2, i2))

        def _merge(a, b):
            a1, ai1, a2, ai2 = a
            b1, bi1, b2, bi2 = b
            swap = b1 > a1
            w1 = jnp.where(swap, b1, a1)
            wi1 = jnp.where(swap, bi1, ai1)
            ca = jnp.where(swap, a1, a2)
            cai = jnp.where(swap, ai1, ai2)
            cb = jnp.where(swap, b2, b1)
            cbi = jnp.where(swap, bi2, bi1)
            take_b = cb > ca
            w2 = jnp.where(take_b, cb, ca)
            wi2 = jnp.where(take_b, cbi, cai)
            return w1, wi1, w2, wi2

        m1, i1, m2, i2 = _merge(_merge(tops[0], tops[1]),
                                _merge(tops[2], tops[3]))

        e2 = jnp.exp(m2 - m1)
        den = 1.0 + e2
        plsc.store_scatter(gv, [trow, i1], 1.0 / den)
        plsc.store_scatter(gv, [trow, i2], e2 / den)
        return carry

    lax.fori_loop(0, _GROUPS_W, _group, 0, unroll=False)

    pltpu.sync_copy(gv, gates_hbm.at[pl.ds(tbase, _ROWS_W), :])


@functools.partial(
    pl.kernel,
    out_type=jax.ShapeDtypeStruct((_T, _E), jnp.float32),
    mesh=plsc.VectorSubcoreMesh(core_axis_name="c", subcore_axis_name="s"),
    scratch_types=[
        pltpu.VMEM((_E, _TILES_W, _LT), jnp.float32),
        pltpu.VMEM((_ROWS_W, _E), jnp.float32),
    ],
    compiler_params=pltpu.CompilerParams(needs_layout_passes=False),
)
def _sc_gates(logits_t_hbm, gates_hbm, lt, gv):
    _sc_gates_body(logits_t_hbm, gates_hbm, lt, gv)


def kernel(x, W_gate, W_noise):
    del W_noise
    logits_t, load = _tc_stage(x, W_gate)
    gates = _sc_gates(logits_t)
    return gates, load


# speedup vs baseline: 1.0290x; 1.0290x over previous
import functools

import jax
import jax.numpy as jnp
from jax import lax
from jax.experimental import pallas as pl
from jax.experimental.pallas import tpu as pltpu
from jax.experimental.pallas import tpu_sc as plsc

_T, _D, _E = 8192, 2048, 64
_NC, _NS, _L = 2, 16, 16
_NW = _NC * _NS
_ROWS_W = _T // _NW
_GROUPS_W = _ROWS_W // _L
_LT = 128
_TILES_W = _ROWS_W // _LT


def _logits_block(x_ref, w_ref, logits_t_ref, load_ref):
    x = x_ref[...]
    w = w_ref[...]
    logits = lax.dot_general(
        x, w, (((1,), (1,)), ((), ())),
        preferred_element_type=jnp.float32)
    bt = logits.shape[0]
    logits_t_ref[...] = logits.T.reshape(_E, bt // _LT, _LT)
    m = jnp.max(logits, axis=1, keepdims=True)
    ex = jnp.exp(logits - m)
    load_ref[...] = ex / jnp.sum(ex, axis=1, keepdims=True)


def _tc_stage(x, w_gate, block_t=1024):
    t, d = x.shape
    e = w_gate.shape[0]
    bt_tiles = block_t // _LT
    return pl.pallas_call(
        _logits_block,
        grid=(t // block_t,),
        in_specs=[
            pl.BlockSpec((block_t, d), lambda i: (i, 0)),
            pl.BlockSpec((e, d), lambda i: (0, 0)),
        ],
        out_specs=[
            pl.BlockSpec((e, bt_tiles, _LT), lambda i: (0, i, 0)),
            pl.BlockSpec((block_t, e), lambda i: (i, 0)),
        ],
        out_shape=[
            jax.ShapeDtypeStruct((e, t // _LT, _LT), jnp.float32),
            jax.ShapeDtypeStruct((t, e), jnp.float32),
        ],
    )(x, w_gate)


def _sc_gates_body(logits_t_hbm, gates_hbm, lt, gv, sem_a, sem_b, sem_o):
    c = lax.axis_index("c")
    s = lax.axis_index("s")
    wid = s * _NC + c
    tbase = wid * _ROWS_W

    cp_a = pltpu.make_async_copy(
        logits_t_hbm.at[:, pl.ds(wid * _TILES_W, 1), :],
        lt.at[:, pl.ds(0, 1), :], sem_a)
    cp_b = pltpu.make_async_copy(
        logits_t_hbm.at[:, pl.ds(wid * _TILES_W + 1, 1), :],
        lt.at[:, pl.ds(1, 1), :], sem_b)
    cp_a.start()
    cp_b.start()
    cp_a.wait()

    zeros = jnp.zeros((_L,), jnp.float32)
    iota = lax.iota(jnp.int32, _L)
    minf = jnp.full((_L,), -jnp.inf, jnp.float32)
    izero = jnp.zeros((_L,), jnp.int32)

    def _group(g, carry):
        t0 = pl.multiple_of(g * _L, _L)
        tile = g // (_LT // _L)
        toff = pl.multiple_of((g % (_LT // _L)) * _L, _L)
        trow = t0 + iota

        for k in range(_L * _E // _L):
            gv[t0 + k // (_E // _L), pl.ds((k % (_E // _L)) * _L, _L)] = zeros

        n_chains = 4
        span = _E // n_chains
        tops = []
        for ch in range(n_chains):
            m1, i1, m2, i2 = minf, izero, minf, izero
            for j in range(span):
                e_i = ch * span + j
                v = lt[e_i, tile, pl.ds(toff, _L)]
                ev = jnp.full((_L,), e_i, jnp.int32)
                gt1 = v > m1
                gt2 = v > m2
                m2n = jnp.where(gt1, m1, jnp.where(gt2, v, m2))
                i2n = jnp.where(gt1, i1, jnp.where(gt2, ev, i2))
                m1 = jnp.where(gt1, v, m1)
                i1 = jnp.where(gt1, ev, i1)
                m2, i2 = m2n, i2n
            tops.append((m1, i1, m2, i2))

        def _merge(a, b):
            a1, ai1, a2, ai2 = a
            b1, bi1, b2, bi2 = b
            swap = b1 > a1
            w1 = jnp.where(swap, b1, a1)
            wi1 = jnp.where(swap, bi1, ai1)
            ca = jnp.where(swap, a1, a2)
            cai = jnp.where(swap, ai1, ai2)
            cb = jnp.where(swap, b2, b1)
            cbi = jnp.where(swap, bi2, bi1)
            take_b = cb > ca
            w2 = jnp.where(take_b, cb, ca)
            wi2 = jnp.where(take_b, cbi, cai)
            return w1, wi1, w2, wi2

        m1, i1, m2, i2 = _merge(_merge(tops[0], tops[1]),
                                _merge(tops[2], tops[3]))

        e2 = jnp.exp(m2 - m1)
        den = 1.0 + e2
        plsc.store_scatter(gv, [trow, i1], 1.0 / den)
        plsc.store_scatter(gv, [trow, i2], e2 / den)
        return carry

    half = _GROUPS_W // 2
    half_rows = _ROWS_W // 2
    lax.fori_loop(0, half, _group, 0, unroll=False)
    cp_o = pltpu.make_async_copy(
        gv.at[pl.ds(0, half_rows), :],
        gates_hbm.at[pl.ds(tbase, half_rows), :], sem_o)
    cp_o.start()
    cp_b.wait()
    lax.fori_loop(half, _GROUPS_W, _group, 0, unroll=False)
    cp_o.wait()
    pltpu.sync_copy(gv.at[pl.ds(half_rows, half_rows), :],
                    gates_hbm.at[pl.ds(tbase + half_rows, half_rows), :])


@functools.partial(
    pl.kernel,
    out_type=jax.ShapeDtypeStruct((_T, _E), jnp.float32),
    mesh=plsc.VectorSubcoreMesh(core_axis_name="c", subcore_axis_name="s"),
    scratch_types=[
        pltpu.VMEM((_E, _TILES_W, _LT), jnp.float32),
        pltpu.VMEM((_ROWS_W, _E), jnp.float32),
        pltpu.SemaphoreType.DMA,
        pltpu.SemaphoreType.DMA,
        pltpu.SemaphoreType.DMA,
    ],
    compiler_params=pltpu.CompilerParams(needs_layout_passes=False),
)
def _sc_gates(logits_t_hbm, gates_hbm, lt, gv, sem_a, sem_b, sem_o):
    _sc_gates_body(logits_t_hbm, gates_hbm, lt, gv, sem_a, sem_b, sem_o)


def kernel(x, W_gate, W_noise):
    del W_noise
    logits_t, load = _tc_stage(x, W_gate)
    gates = _sc_gates(logits_t)
    return gates, load
